# speedup vs baseline: 2.9689x; 2.9689x over previous
import functools

import jax
import jax.numpy as jnp
from jax import lax
from jax.experimental import pallas as pl
from jax.experimental.pallas import tpu as pltpu
from jax.experimental.pallas import tpu_sc as plsc

VOCAB = 100000
EMB_DIM = 128
BATCH = 4096
HIST = 50

NUM_CORES = 2
NUM_SUBCORES = 16
NUM_WORKERS = NUM_CORES * NUM_SUBCORES
TOTAL_ROWS = BATCH * HIST
ROWS_PER_WORKER = TOTAL_ROWS // NUM_WORKERS
CHUNK = 128
NCHUNKS = ROWS_PER_WORKER // CHUNK


def _emb_body(idx_hbm, table_hbm, out_hbm, idx_v, buf, sem):
    wid = lax.axis_index("s") * NUM_CORES + lax.axis_index("c")
    pltpu.sync_copy(idx_hbm.at[wid], idx_v)
    base = wid * ROWS_PER_WORKER

    def chunk(j, carry):
        pltpu.async_copy(table_hbm.at[idx_v.at[j]], buf, sem).wait()
        pltpu.sync_copy(buf, out_hbm.at[pl.ds(base + j * CHUNK, CHUNK)])
        return carry

    lax.fori_loop(0, NCHUNKS, chunk, 0)


@jax.jit
def _emb_call(idx, weight):
    mesh = plsc.VectorSubcoreMesh(
        core_axis_name="c", subcore_axis_name="s",
        num_cores=NUM_CORES, num_subcores=NUM_SUBCORES,
    )
    run = pl.kernel(
        _emb_body,
        out_type=jax.ShapeDtypeStruct((TOTAL_ROWS, EMB_DIM), jnp.float32),
        mesh=mesh,
        scratch_types=[
            pltpu.VMEM((NCHUNKS, CHUNK), jnp.int32),
            pltpu.VMEM((CHUNK, EMB_DIM), jnp.float32),
            pltpu.SemaphoreType.DMA,
        ],
    )
    return run(idx, weight)


def kernel(input, weight):
    idx = input.astype(jnp.int32).reshape(NUM_WORKERS, NCHUNKS, CHUNK)
    out = _emb_call(idx, weight)
    return out.reshape(BATCH, HIST, EMB_DIM)


# speedup vs baseline: 3.3413x; 1.1254x over previous
import jax
import jax.numpy as jnp
from jax import lax
from jax.experimental import pallas as pl
from jax.experimental.pallas import tpu as pltpu
from jax.experimental.pallas import tpu_sc as plsc

VOCAB = 100000
EMB_DIM = 128
BATCH = 4096
HIST = 50

NUM_CORES = 2
NUM_SUBCORES = 16
NUM_WORKERS = NUM_CORES * NUM_SUBCORES
TOTAL_ROWS = BATCH * HIST
ROWS_PER_WORKER = TOTAL_ROWS // NUM_WORKERS
CHUNK = 128
NCHUNKS = ROWS_PER_WORKER // CHUNK
NBUF = 5
LOOK = NBUF - 1


def _emb_body(idx_hbm, table_hbm, out_hbm, idx_v, *bufs_and_sems):
    bufs = bufs_and_sems[:NBUF]
    gsems = bufs_and_sems[NBUF:2 * NBUF]
    ssems = bufs_and_sems[2 * NBUF:3 * NBUF]

    wid = lax.axis_index("s") * NUM_CORES + lax.axis_index("c")
    pltpu.sync_copy(idx_hbm.at[wid], idx_v)
    base = wid * ROWS_PER_WORKER

    def fire_gather(c, b):
        pltpu.async_copy(table_hbm.at[idx_v.at[c]], bufs[b], gsems[b])

    def wait_gather(c, b):
        pltpu.make_async_copy(table_hbm.at[idx_v.at[c]], bufs[b], gsems[b]).wait()

    def fire_scatter(c, b):
        pltpu.async_copy(bufs[b], out_hbm.at[pl.ds(base + c * CHUNK, CHUNK)], ssems[b])

    def wait_scatter(b):
        pltpu.make_async_copy(bufs[b], out_hbm.at[pl.ds(base, CHUNK)], ssems[b]).wait()

    for b in range(LOOK):
        fire_gather(b, b)

    wait_gather(0, 0)
    fire_scatter(0, 0)
    fire_gather(LOOK, LOOK % NBUF)

    def outer(g, carry):
        for bp in range(NBUF):
            c = g * NBUF + 1 + bp
            b = (bp + 1) % NBUF
            tb = bp
            wait_gather(c, b)
            fire_scatter(c, b)
            wait_scatter(tb)
            fire_gather(c + LOOK, tb)
        return carry

    lax.fori_loop(0, (NCHUNKS - LOOK - 1) // NBUF, outer, 0)

    for c in range(NCHUNKS - LOOK, NCHUNKS):
        b = c % NBUF
        wait_gather(c, b)
        fire_scatter(c, b)
    for b in range(NBUF):
        wait_scatter(b)


@jax.jit
def _emb_call(idx, weight):
    mesh = plsc.VectorSubcoreMesh(
        core_axis_name="c", subcore_axis_name="s",
        num_cores=NUM_CORES, num_subcores=NUM_SUBCORES,
    )
    run = pl.kernel(
        _emb_body,
        out_type=jax.ShapeDtypeStruct((TOTAL_ROWS, EMB_DIM), jnp.float32),
        mesh=mesh,
        scratch_types=(
            [pltpu.VMEM((NCHUNKS, CHUNK), jnp.int32)]
            + [pltpu.VMEM((CHUNK, EMB_DIM), jnp.float32) for _ in range(NBUF)]
            + [pltpu.SemaphoreType.DMA for _ in range(2 * NBUF)]
        ),
    )
    return run(idx, weight)


def kernel(input, weight):
    idx = input.astype(jnp.int32).reshape(NUM_WORKERS, NCHUNKS, CHUNK)
    out = _emb_call(idx, weight)
    return out.reshape(BATCH, HIST, EMB_DIM)
